# baseline (speedup 1.0000x reference)
import functools

import jax
import jax.numpy as jnp
from jax import lax
from jax.experimental import pallas as pl
from jax.experimental.pallas import tpu as pltpu
from jax.experimental.pallas import tpu_sc as plsc

BATCH = 16384
NUM_ROWS = 100000
BLK = 128
NBLOCKS = 784
FLAT_WORDS = NBLOCKS * 2 * BLK
NUM_CORES = 2
NUM_SUBCORES = 16
NUM_WORKERS = NUM_CORES * NUM_SUBCORES
PER_WORKER = BATCH // NUM_WORKERS
CHUNK = 128
K = PER_WORKER // CHUNK
L = 16


def _gather_kernel(idx_hbm, flat_hbm, out_hbm, idx_v, off0_v, off1_v,
                   c0_v, c1_v, spmem, sem, ssem, osem):
    c = lax.axis_index("c")
    s = lax.axis_index("s")
    wid = s * NUM_CORES + c
    seg = FLAT_WORDS // NUM_SUBCORES
    stage = pltpu.async_copy(
        flat_hbm.at[pl.ds(s * seg, seg)], spmem.at[pl.ds(s * seg, seg)], ssem)
    pltpu.sync_copy(idx_hbm.at[wid], idx_v)
    for i in range(PER_WORKER // L):
        r = idx_v[pl.ds(i * L, L)]
        off = (jnp.left_shift(jnp.right_shift(r, 7), 8)
               + jnp.bitwise_and(r, BLK - 1))
        off0_v[pl.ds(i * L, L)] = off
        off1_v[pl.ds(i * L, L)] = off + BLK
    stage.wait()
    plsc.subcore_barrier()
    g0 = pltpu.async_copy(spmem.at[off0_v], c0_v, sem)
    g1 = pltpu.async_copy(spmem.at[off1_v], c1_v, osem)
    outs = []
    g0.wait()
    for j in range(K):
        sl = pl.ds(j * CHUNK, CHUNK)
        outs.append(
            pltpu.async_copy(c0_v.at[sl], out_hbm.at[wid * K + j, 0], sem))
    g1.wait()
    for j in range(K):
        sl = pl.ds(j * CHUNK, CHUNK)
        outs.append(
            pltpu.async_copy(c1_v.at[sl], out_hbm.at[wid * K + j, 1], osem))
    for o in outs:
        o.wait()


@jax.jit
def _lookup(participant, param_tensor):
    idx2d = participant.reshape(NUM_WORKERS, PER_WORKER)
    padded = jnp.pad(param_tensor, ((0, NBLOCKS * BLK - NUM_ROWS), (0, 0)))
    flat = padded.reshape(NBLOCKS, BLK, 2).transpose(0, 2, 1).reshape(-1)
    mesh = plsc.VectorSubcoreMesh(core_axis_name="c", subcore_axis_name="s")
    run = functools.partial(
        pl.kernel,
        mesh=mesh,
        out_type=jax.ShapeDtypeStruct((BATCH // BLK, 2, BLK), jnp.float32),
        scratch_types=[
            pltpu.VMEM((PER_WORKER,), jnp.int32),
            pltpu.VMEM((PER_WORKER,), jnp.int32),
            pltpu.VMEM((PER_WORKER,), jnp.int32),
            pltpu.VMEM((PER_WORKER,), jnp.float32),
            pltpu.VMEM((PER_WORKER,), jnp.float32),
            pltpu.VMEM_SHARED((FLAT_WORDS,), jnp.float32),
            pltpu.SemaphoreType.DMA,
            pltpu.SemaphoreType.DMA,
            pltpu.SemaphoreType.DMA,
        ],
        compiler_params=pltpu.CompilerParams(use_tc_tiling_on_sc=False),
    )(_gather_kernel)
    out3 = run(idx2d, flat)
    return out3.transpose(0, 2, 1).reshape(BATCH, 2)


def kernel(participant, param_tensor):
    return _lookup(participant, param_tensor)
